# baseline (speedup 1.0000x reference)
import functools

import jax
import jax.numpy as jnp
from jax import lax
from jax.experimental import pallas as pl
from jax.experimental.pallas import tpu as pltpu
from jax.experimental.pallas import tpu_sc as plsc

_D = 128
_K = 16
_NG = 8
_NQ = 4
_KQ = _K // _NQ


def _tables_body(xu_ref, xi_ref, ei_ref, wpu_ref, bpu_ref, wpi_ref, bpi_ref,
                 ws_ref, bs_ref, lt_ref, flat_ref):
    dn = (((0,), (1,)), ((), ()))
    hp = lax.Precision.HIGHEST
    wsu = ws_ref[0:_D, :]
    wsi = ws_ref[_D:2 * _D, :]
    wu = jnp.dot(wpu_ref[:], wsu, preferred_element_type=jnp.float32,
                 precision=hp)
    cu = jnp.dot(bpu_ref[:], wsu, preferred_element_type=jnp.float32,
                 precision=hp)
    lut = (lax.dot_general(wu, xu_ref[:], dn,
                           preferred_element_type=jnp.float32, precision=hp)
           + cu.reshape(_K, 1))
    wi = jnp.dot(wpi_ref[:], wsi, preferred_element_type=jnp.float32,
                 precision=hp)
    ci = (jnp.dot(bpi_ref[:], wsi, preferred_element_type=jnp.float32,
                  precision=hp)
          + bs_ref[:])
    lit = (lax.dot_general(wi, xi_ref[:], dn,
                           preferred_element_type=jnp.float32, precision=hp)
           + ci.reshape(_K, 1))
    for q in range(_NQ):
        lt_ref[q, :, :] = lut[q * _KQ:(q + 1) * _KQ, :]
        lt_ref[_NQ + q, :, :] = lit[q * _KQ:(q + 1) * _KQ, :]
    e = ei_ref.shape[1]
    flat_ref[pl.ds(0, e)] = ei_ref[0, :]
    flat_ref[pl.ds(e, e)] = ei_ref[1, :]


def _compute_tables(x_user, x_item, edge_index, wpu, bpu, wpi, bpi, ws, bs):
    n = x_user.shape[0]
    assert x_item.shape[0] == n
    e = edge_index.shape[1]
    return pl.pallas_call(
        _tables_body,
        out_shape=(
            jax.ShapeDtypeStruct((2 * _NQ, _KQ, n), jnp.float32),
            jax.ShapeDtypeStruct((2 * e,), jnp.int32),
        ),
    )(x_user, x_item, edge_index, wpu, bpu.reshape(1, _D),
      wpi, bpi.reshape(1, _D), ws, bs.reshape(1, _K))


def _make_sc_gather(n, E, C):
    info = plsc.get_sparse_core_info()
    NC, NS = info.num_cores, info.num_subcores
    assert NC * NS == 32 and C % 128 == 0 and E % C == 0
    n_chunks = E // C
    mesh = plsc.VectorSubcoreMesh(core_axis_name="c", subcore_axis_name="s")

    @functools.partial(
        pl.kernel,
        out_type=jax.ShapeDtypeStruct((_NQ, _KQ, E), jnp.float32),
        mesh=mesh,
        compiler_params=pltpu.CompilerParams(needs_layout_passes=False),
        scratch_types=[
            pltpu.VMEM((_KQ, n), jnp.float32),
            pltpu.VMEM((_KQ, n), jnp.float32),
            pltpu.VMEM((C,), jnp.int32),
            pltpu.VMEM((C,), jnp.int32),
            pltpu.VMEM((C,), jnp.int32),
            pltpu.VMEM((C,), jnp.int32),
            pltpu.VMEM((_KQ, C), jnp.float32),
            pltpu.VMEM((_KQ, C), jnp.float32),
            pltpu.SemaphoreType.DMA,
            pltpu.SemaphoreType.DMA,
            pltpu.SemaphoreType.DMA,
            pltpu.SemaphoreType.DMA,
            pltpu.SemaphoreType.DMA,
            pltpu.SemaphoreType.DMA,
        ],
    )
    def sc_gather(srcdst_hbm, lt_hbm, out_hbm, tab_u, tab_i,
                  src0, src1, dst0, dst1, out0, out1,
                  us0, us1, ds0, ds1, os0, os1):
        wid = lax.axis_index("s") * NC + lax.axis_index("c")
        g = wid // _NQ
        q = wid % _NQ
        pltpu.sync_copy(lt_hbm.at[q], tab_u)
        pltpu.sync_copy(lt_hbm.at[_NQ + q], tab_i)

        full_slots = n_chunks // _NG
        rem = n_chunks % _NG
        total_slots = full_slots + (1 if rem else 0)
        src_bufs = (src0, src1)
        dst_bufs = (dst0, dst1)
        out_bufs = (out0, out1)
        usems = (us0, us1)
        dsems = (ds0, ds1)
        osems = (os0, os1)

        def slot_base(s):
            return (s * _NG + g) * C

        def src_copy(s):
            return pltpu.make_async_copy(
                srcdst_hbm.at[pl.ds(slot_base(s), C)],
                src_bufs[s % 2], usems[s % 2])

        def dst_copy(s):
            return pltpu.make_async_copy(
                srcdst_hbm.at[pl.ds(E + slot_base(s), C)],
                dst_bufs[s % 2], dsems[s % 2])

        def out_copy(s):
            return pltpu.make_async_copy(
                out_bufs[s % 2],
                out_hbm.at[q, :, pl.ds(slot_base(s), C)],
                osems[s % 2])

        def compute(s):
            src = src_bufs[s % 2]
            dst = dst_bufs[s % 2]
            out_v = out_bufs[s % 2]

            @plsc.parallel_loop(0, C, step=16, unroll=4)
            def vec_body(o):
                eu = src[pl.ds(o, 16)]
                ei = dst[pl.ds(o, 16)]
                for k in range(_KQ):
                    ks = jnp.full((16,), k, jnp.int32)
                    out_v[k, pl.ds(o, 16)] = (
                        plsc.load_gather(tab_u, [ks, eu])
                        + plsc.load_gather(tab_i, [ks, ei]))

        def maybe_guard(s, fn):
            if s >= full_slots:
                pl.when(g < rem)(fn)
            else:
                fn()

        def start_idx(s):
            def go():
                src_copy(s).start()
                dst_copy(s).start()
            return go

        maybe_guard(0, start_idx(0))
        for s in range(total_slots):
            def slot_work(s=s):
                if s + 1 < total_slots:
                    maybe_guard(s + 1, start_idx(s + 1))
                src_copy(s).wait()
                dst_copy(s).wait()
                if s - 2 >= 0:
                    out_copy(s - 2).wait()
                compute(s)
                out_copy(s).start()

            maybe_guard(s, slot_work)

        if rem and full_slots >= 2:
            pl.when(g >= rem)(lambda: out_copy(full_slots - 2).wait())
        if full_slots >= 1:
            out_copy(full_slots - 1).wait()
        if rem:
            pl.when(g < rem)(lambda: out_copy(full_slots).wait())

    return sc_gather


def _combine_body(logits_ref, anchors_ref, out_ref):
    ls = [logits_ref[qq] for qq in range(_NQ)]
    ls = [jnp.where(l >= 0, l, 0.01 * l) for l in ls]
    m = jnp.max(jnp.maximum(jnp.maximum(ls[0], ls[1]),
                            jnp.maximum(ls[2], ls[3])),
                axis=0, keepdims=True)
    es = [jnp.exp(l - m) for l in ls]
    s = (jnp.sum(es[0], axis=0, keepdims=True)
         + jnp.sum(es[1], axis=0, keepdims=True)
         + jnp.sum(es[2], axis=0, keepdims=True)
         + jnp.sum(es[3], axis=0, keepdims=True))
    r = 1.0 / s
    dn = (((0,), (0,)), ((), ()))
    acc = lax.dot_general(es[0] * r, anchors_ref[0:_KQ, :], dn,
                          preferred_element_type=jnp.float32)
    for qq in range(1, _NQ):
        acc = acc + lax.dot_general(
            es[qq] * r, anchors_ref[qq * _KQ:(qq + 1) * _KQ, :], dn,
            preferred_element_type=jnp.float32)
    out_ref[:] = acc


def _combine(logits3, anchors, block_e):
    E = logits3.shape[2]
    grid = E // block_e
    return pl.pallas_call(
        _combine_body,
        grid=(grid,),
        in_specs=[
            pl.BlockSpec((_NQ, _KQ, block_e), lambda i: (0, 0, i)),
            pl.BlockSpec((_K, _D), lambda i: (0, 0)),
        ],
        out_specs=pl.BlockSpec((block_e, _D), lambda i: (i, 0)),
        out_shape=jax.ShapeDtypeStruct((E, _D), jnp.float32),
    )(logits3, anchors)


def kernel(x_user, x_item, edge_index, W_proj_user, b_proj_user,
           W_proj_item, b_proj_item, W_score, b_score, anchors):
    E = edge_index.shape[1]
    lt, srcdst = _compute_tables(x_user, x_item, edge_index,
                                 W_proj_user, b_proj_user,
                                 W_proj_item, b_proj_item, W_score, b_score)
    logits3 = _make_sc_gather(x_user.shape[0], E, 2560)(srcdst, lt)
    return _combine(logits3, anchors, 2560)


# speedup vs baseline: 1.7879x; 1.7879x over previous
import functools

import jax
import jax.numpy as jnp
from jax import lax
from jax.experimental import pallas as pl
from jax.experimental.pallas import tpu as pltpu
from jax.experimental.pallas import tpu_sc as plsc

_D = 128
_K = 16


def _tables_body(xu_ref, xi_ref, ei_ref, wpu_ref, bpu_ref, wpi_ref, bpi_ref,
                 ws_ref, bs_ref, lt_ref, flat_ref):
    dn = (((0,), (1,)), ((), ()))
    hp = lax.Precision.HIGHEST
    wsu = ws_ref[0:_D, :]
    wsi = ws_ref[_D:2 * _D, :]
    wu = jnp.dot(wpu_ref[:], wsu, preferred_element_type=jnp.float32,
                 precision=hp)
    cu = jnp.dot(bpu_ref[:], wsu, preferred_element_type=jnp.float32,
                 precision=hp)
    lt_ref[0:_K, :] = (lax.dot_general(wu, xu_ref[:], dn,
                                       preferred_element_type=jnp.float32,
                                       precision=hp)
                       + cu.reshape(_K, 1))
    wi = jnp.dot(wpi_ref[:], wsi, preferred_element_type=jnp.float32,
                 precision=hp)
    ci = (jnp.dot(bpi_ref[:], wsi, preferred_element_type=jnp.float32,
                  precision=hp)
          + bs_ref[:])
    lt_ref[_K:2 * _K, :] = (lax.dot_general(wi, xi_ref[:], dn,
                                            preferred_element_type=jnp.float32,
                                            precision=hp)
                            + ci.reshape(_K, 1))
    e = ei_ref.shape[1]
    flat_ref[pl.ds(0, e)] = ei_ref[0, :]
    flat_ref[pl.ds(e, e)] = ei_ref[1, :]


def _compute_tables(x_user, x_item, edge_index, wpu, bpu, wpi, bpi, ws, bs):
    n = x_user.shape[0]
    assert x_item.shape[0] == n
    e = edge_index.shape[1]
    return pl.pallas_call(
        _tables_body,
        out_shape=(
            jax.ShapeDtypeStruct((2 * _K, n), jnp.float32),
            jax.ShapeDtypeStruct((2 * e,), jnp.int32),
        ),
    )(x_user, x_item, edge_index, wpu, bpu.reshape(1, _D),
      wpi, bpi.reshape(1, _D), ws, bs.reshape(1, _K))


_NG = 8
_KH = _K // 2


def _make_sc_gather(n, E, C):
    info = plsc.get_sparse_core_info()
    NC, NS = info.num_cores, info.num_subcores
    assert NC * NS == 32 and C % 128 == 0 and E % C == 0
    n_chunks = E // C
    mesh = plsc.VectorSubcoreMesh(core_axis_name="c", subcore_axis_name="s")

    @functools.partial(
        pl.kernel,
        out_type=jax.ShapeDtypeStruct((2 * _K, E), jnp.float32),
        mesh=mesh,
        compiler_params=pltpu.CompilerParams(needs_layout_passes=False),
        scratch_types=[
            pltpu.VMEM((_KH, n), jnp.float32),
            pltpu.VMEM((C,), jnp.int32),
            pltpu.VMEM((C,), jnp.int32),
            pltpu.VMEM((_KH, C), jnp.float32),
            pltpu.VMEM((_KH, C), jnp.float32),
            pltpu.SemaphoreType.DMA,
            pltpu.SemaphoreType.DMA,
            pltpu.SemaphoreType.DMA,
            pltpu.SemaphoreType.DMA,
        ],
    )
    def sc_gather(srcdst_hbm, lt_hbm, out_hbm, tab, idx0, idx1,
                  out0, out1, isem0, isem1, osem0, osem1):
        wid = lax.axis_index("s") * NC + lax.axis_index("c")
        g = wid // 4
        h = (wid >> 1) & 1
        t = wid & 1
        row0 = t * _K + h * _KH
        pltpu.sync_copy(lt_hbm.at[pl.ds(row0, _KH), :], tab)
        full_slots = n_chunks // _NG
        rem = n_chunks % _NG
        total_slots = full_slots + (1 if rem else 0)
        idx_bufs = (idx0, idx1)
        out_bufs = (out0, out1)
        isems = (isem0, isem1)
        osems = (osem0, osem1)

        def slot_base(s):
            return (s * _NG + g) * C

        def idx_copy(s):
            return pltpu.make_async_copy(
                srcdst_hbm.at[pl.ds(t * E + slot_base(s), C)],
                idx_bufs[s % 2], isems[s % 2])

        def out_copy(s):
            return pltpu.make_async_copy(
                out_bufs[s % 2],
                out_hbm.at[pl.ds(row0, _KH), pl.ds(slot_base(s), C)],
                osems[s % 2])

        def compute(s):
            idx = idx_bufs[s % 2]
            out_v = out_bufs[s % 2]

            @plsc.parallel_loop(0, C, step=16, unroll=4)
            def vec_body(o):
                e = idx[pl.ds(o, 16)]
                for k in range(_KH):
                    ks = jnp.full((16,), k, jnp.int32)
                    out_v[k, pl.ds(o, 16)] = plsc.load_gather(tab, [ks, e])

        def maybe_guard(s, fn):
            if s >= full_slots:
                pl.when(g < rem)(fn)
            else:
                fn()

        maybe_guard(0, lambda: idx_copy(0).start())
        for s in range(total_slots):
            def slot_work(s=s):
                if s + 1 < total_slots:
                    maybe_guard(s + 1, lambda: idx_copy(s + 1).start())
                idx_copy(s).wait()
                if s - 2 >= 0:
                    out_copy(s - 2).wait()
                compute(s)
                out_copy(s).start()

            maybe_guard(s, slot_work)

        if rem and full_slots >= 2:
            pl.when(g >= rem)(lambda: out_copy(full_slots - 2).wait())
        if full_slots >= 1:
            out_copy(full_slots - 1).wait()
        if rem:
            pl.when(g < rem)(lambda: out_copy(full_slots).wait())

    return sc_gather


def _combine_body(logits_ref, anchors_ref, out_ref):
    l = logits_ref[0:_K, :] + logits_ref[_K:2 * _K, :]
    l = jnp.where(l >= 0, l, 0.01 * l)
    m = jnp.max(l, axis=0, keepdims=True)
    e = jnp.exp(l - m)
    s = jnp.sum(e, axis=0, keepdims=True)
    out_ref[:] = lax.dot_general(e / s, anchors_ref[:],
                                 (((0,), (0,)), ((), ())),
                                 preferred_element_type=jnp.float32)


def _combine(logits_t, anchors, block_e):
    E = logits_t.shape[1]
    grid = E // block_e
    return pl.pallas_call(
        _combine_body,
        grid=(grid,),
        in_specs=[
            pl.BlockSpec((2 * _K, block_e), lambda i: (0, i)),
            pl.BlockSpec((_K, _D), lambda i: (0, 0)),
        ],
        out_specs=pl.BlockSpec((block_e, _D), lambda i: (i, 0)),
        out_shape=jax.ShapeDtypeStruct((E, _D), jnp.float32),
    )(logits_t, anchors)


def kernel(x_user, x_item, edge_index, W_proj_user, b_proj_user,
           W_proj_item, b_proj_item, W_score, b_score, anchors):
    E = edge_index.shape[1]
    lt, srcdst = _compute_tables(x_user, x_item, edge_index,
                                 W_proj_user, b_proj_user,
                                 W_proj_item, b_proj_item, W_score, b_score)
    logits_t = _make_sc_gather(x_user.shape[0], E, 2560)(srcdst, lt)
    return _combine(logits_t, anchors, 6400)


# speedup vs baseline: 1.9805x; 1.1077x over previous
import functools

import jax
import jax.numpy as jnp
from jax import lax
from jax.experimental import pallas as pl
from jax.experimental.pallas import tpu as pltpu
from jax.experimental.pallas import tpu_sc as plsc

_D = 128
_K = 16


def _tables_body(xu_ref, xi_ref, ei_ref, wpu_ref, bpu_ref, wpi_ref, bpi_ref,
                 ws_ref, bs_ref, lt_ref, flat_ref):
    dn = (((0,), (1,)), ((), ()))
    hp = lax.Precision.HIGHEST
    wsu = ws_ref[0:_D, :]
    wsi = ws_ref[_D:2 * _D, :]
    wu = jnp.dot(wpu_ref[:], wsu, preferred_element_type=jnp.float32,
                 precision=hp)
    cu = jnp.dot(bpu_ref[:], wsu, preferred_element_type=jnp.float32,
                 precision=hp)
    lt_ref[0:_K, :] = (lax.dot_general(wu, xu_ref[:], dn,
                                       preferred_element_type=jnp.float32,
                                       precision=hp)
                       + cu.reshape(_K, 1))
    wi = jnp.dot(wpi_ref[:], wsi, preferred_element_type=jnp.float32,
                 precision=hp)
    ci = (jnp.dot(bpi_ref[:], wsi, preferred_element_type=jnp.float32,
                  precision=hp)
          + bs_ref[:])
    lt_ref[_K:2 * _K, :] = (lax.dot_general(wi, xi_ref[:], dn,
                                            preferred_element_type=jnp.float32,
                                            precision=hp)
                            + ci.reshape(_K, 1))
    e = ei_ref.shape[1]
    flat_ref[pl.ds(0, e)] = ei_ref[0, :]
    flat_ref[pl.ds(e, e)] = ei_ref[1, :]


def _compute_tables(x_user, x_item, edge_index, wpu, bpu, wpi, bpi, ws, bs):
    n = x_user.shape[0]
    assert x_item.shape[0] == n
    e = edge_index.shape[1]
    return pl.pallas_call(
        _tables_body,
        out_shape=(
            jax.ShapeDtypeStruct((2 * _K, n), jnp.float32),
            jax.ShapeDtypeStruct((2 * e,), jnp.int32),
        ),
    )(x_user, x_item, edge_index, wpu, bpu.reshape(1, _D),
      wpi, bpi.reshape(1, _D), ws, bs.reshape(1, _K))


_NG = 8
_KH = _K // 2


def _make_sc_gather(n, E, C):
    info = plsc.get_sparse_core_info()
    NC, NS = info.num_cores, info.num_subcores
    assert NC * NS == 32 and C % 128 == 0 and E % C == 0
    n_chunks = E // C
    mesh = plsc.VectorSubcoreMesh(core_axis_name="c", subcore_axis_name="s")

    @functools.partial(
        pl.kernel,
        out_type=jax.ShapeDtypeStruct((2 * _K, E), jnp.float32),
        mesh=mesh,
        compiler_params=pltpu.CompilerParams(needs_layout_passes=False),
        scratch_types=[
            pltpu.VMEM((_KH, n), jnp.float32),
            pltpu.VMEM((C,), jnp.int32),
            pltpu.VMEM((C,), jnp.int32),
            pltpu.VMEM((_KH, C), jnp.float32),
            pltpu.VMEM((_KH, C), jnp.float32),
            pltpu.SemaphoreType.DMA,
            pltpu.SemaphoreType.DMA,
            pltpu.SemaphoreType.DMA,
            pltpu.SemaphoreType.DMA,
        ],
    )
    def sc_gather(srcdst_hbm, lt_hbm, out_hbm, tab, idx0, idx1,
                  out0, out1, isem0, isem1, osem0, osem1):
        wid = lax.axis_index("s") * NC + lax.axis_index("c")
        g = wid // 4
        h = (wid >> 1) & 1
        t = wid & 1
        row0 = t * _K + h * _KH
        pltpu.sync_copy(lt_hbm.at[pl.ds(row0, _KH), :], tab)
        full_slots = n_chunks // _NG
        rem = n_chunks % _NG
        total_slots = full_slots + (1 if rem else 0)
        idx_bufs = (idx0, idx1)
        out_bufs = (out0, out1)
        isems = (isem0, isem1)
        osems = (osem0, osem1)

        def slot_base(s):
            return (s * _NG + g) * C

        def idx_copy(s):
            return pltpu.make_async_copy(
                srcdst_hbm.at[pl.ds(t * E + slot_base(s), C)],
                idx_bufs[s % 2], isems[s % 2])

        def out_copy(s):
            return pltpu.make_async_copy(
                out_bufs[s % 2],
                out_hbm.at[pl.ds(row0, _KH), pl.ds(slot_base(s), C)],
                osems[s % 2])

        def compute(s):
            idx = idx_bufs[s % 2]
            out_v = out_bufs[s % 2]

            @plsc.parallel_loop(0, C, step=16, unroll=4)
            def vec_body(o):
                e = idx[pl.ds(o, 16)]
                for k in range(_KH):
                    ks = jnp.full((16,), k, jnp.int32)
                    out_v[k, pl.ds(o, 16)] = plsc.load_gather(tab, [ks, e])

        def maybe_guard(s, fn):
            if s >= full_slots:
                pl.when(g < rem)(fn)
            else:
                fn()

        maybe_guard(0, lambda: idx_copy(0).start())
        for s in range(total_slots):
            def slot_work(s=s):
                if s + 1 < total_slots:
                    maybe_guard(s + 1, lambda: idx_copy(s + 1).start())
                idx_copy(s).wait()
                if s - 2 >= 0:
                    out_copy(s - 2).wait()
                compute(s)
                out_copy(s).start()

            maybe_guard(s, slot_work)

        if rem and full_slots >= 2:
            pl.when(g >= rem)(lambda: out_copy(full_slots - 2).wait())
        if full_slots >= 1:
            out_copy(full_slots - 1).wait()
        if rem:
            pl.when(g < rem)(lambda: out_copy(full_slots).wait())

    return sc_gather


def _combine_body(logits_ref, anchors_ref, out_ref):
    l = logits_ref[0:_K, :] + logits_ref[_K:2 * _K, :]
    l = jnp.where(l >= 0, l, 0.01 * l)
    m = jnp.max(l, axis=0, keepdims=True)
    e = jnp.exp(l - m)
    s = jnp.sum(e, axis=0, keepdims=True)
    out_ref[:] = lax.dot_general(e / s, anchors_ref[:],
                                 (((0,), (0,)), ((), ())),
                                 preferred_element_type=jnp.float32)


def _combine(logits_t, anchors, block_e):
    E = logits_t.shape[1]
    grid = E // block_e
    return pl.pallas_call(
        _combine_body,
        grid=(grid,),
        in_specs=[
            pl.BlockSpec((2 * _K, block_e), lambda i: (0, i)),
            pl.BlockSpec((_K, _D), lambda i: (0, 0)),
        ],
        out_specs=pl.BlockSpec((block_e, _D), lambda i: (i, 0)),
        out_shape=jax.ShapeDtypeStruct((E, _D), jnp.float32),
    )(logits_t, anchors)


def kernel(x_user, x_item, edge_index, W_proj_user, b_proj_user,
           W_proj_item, b_proj_item, W_score, b_score, anchors):
    E = edge_index.shape[1]
    lt, srcdst = _compute_tables(x_user, x_item, edge_index,
                                 W_proj_user, b_proj_user,
                                 W_proj_item, b_proj_item, W_score, b_score)
    logits_t = _make_sc_gather(x_user.shape[0], E, 2560)(srcdst, lt)
    return _combine(logits_t, anchors, 12800)
